# Initial kernel scaffold; baseline (speedup 1.0000x reference)
import jax
import jax.numpy as jnp
from jax.experimental import pallas as pl


def kernel(ch1, ch2, CP_locs, CP_idx):
    raise NotImplementedError("write your pallas kernel here")


# speedup vs baseline: 42.2644x; 42.2644x over previous
import jax
import jax.numpy as jnp
from jax import lax
from jax.experimental import pallas as pl
from jax.experimental.pallas import tpu as pltpu
from jax.experimental.pallas import tpu_sc as plsc

G = 2048
N_TOTAL = 1000000
NC = 2
NS = 16
NW = NC * NS

CHUNK = 1008
GROUPS = CHUNK // 16
NCHUNKS = 31
PER_W = CHUNK * NCHUNKS
TAIL = N_TOTAL - PER_W * NW
TAIL_GROUPS = TAIL // 16
IDX_ROWS = GROUPS * 4


def _cr_weights(t):
    t2 = t * t
    t3 = t2 * t
    w0 = 0.5 * (-t3 + 2.0 * t2 - t)
    w1 = 0.5 * (3.0 * t3 - 5.0 * t2 + 2.0)
    w2 = 0.5 * (-3.0 * t3 + 4.0 * t2 + t)
    w3 = 0.5 * (t3 - t2)
    return w0, w1, w2, w3


def _body(c1x_hbm, c1y_hbm, c2x_hbm, c2y_hbm, rr_hbm, cc_hbm, tab_hbm,
          out_hbm,
          c1x_v, c1y_v, c2x_v, c2y_v, rr_v, cc_v, idx_v, rows_v, out_v, sem):
    cid = lax.axis_index("c")
    sid = lax.axis_index("s")
    wid = sid * NC + cid

    def process_chunk(base, ngroups, acc):
        npts = ngroups * 16
        pltpu.sync_copy(rr_hbm.at[pl.ds(base, npts)], rr_v.at[pl.ds(0, npts)])
        pltpu.sync_copy(cc_hbm.at[pl.ds(base, npts)], cc_v.at[pl.ds(0, npts)])
        pltpu.sync_copy(c2x_hbm.at[pl.ds(base, npts)], c2x_v.at[pl.ds(0, npts)])
        pltpu.sync_copy(c2y_hbm.at[pl.ds(base, npts)], c2y_v.at[pl.ds(0, npts)])
        pltpu.sync_copy(c1x_hbm.at[pl.ds(base, npts)], c1x_v.at[pl.ds(0, npts)])
        pltpu.sync_copy(c1y_hbm.at[pl.ds(base, npts)], c1y_v.at[pl.ds(0, npts)])

        def build_one(g, carry):
            r = rr_v[pl.ds(g * 16, 16)]
            c = cc_v[pl.ds(g * 16, 16)]
            base2 = (r * G + c) * 2
            for i in range(4):
                for j in range(4):
                    off = ((i - 1) * G + (j - 1)) * 2
                    slot = 128 * i + 32 * j
                    idx_v[pl.ds(g * 512 + slot, 16)] = base2 + off
                    idx_v[pl.ds(g * 512 + slot + 16, 16)] = base2 + (off + 1)
            return carry

        lax.fori_loop(0, ngroups, build_one, 0, unroll=False)

        nwords = ngroups * 512
        pltpu.make_async_copy(
            tab_hbm.at[idx_v.at[pl.ds(0, nwords)]],
            rows_v.at[pl.ds(0, nwords)], sem).start()
        pltpu.make_async_copy(
            tab_hbm.at[idx_v.at[pl.ds(0, nwords)]],
            rows_v.at[pl.ds(0, nwords)], sem).wait()

        def comp_one(g, a):
            x = c2x_v[pl.ds(g * 16, 16)]
            y = c2y_v[pl.ds(g * 16, 16)]
            x = lax.rem(x, jnp.float32(1.0))
            y = lax.rem(y, jnp.float32(1.0))
            wx = _cr_weights(x)
            wy = _cr_weights(y)
            mx = jnp.zeros((16,), jnp.float32)
            my = jnp.zeros((16,), jnp.float32)
            for i in range(4):
                rx = jnp.zeros((16,), jnp.float32)
                ry = jnp.zeros((16,), jnp.float32)
                for j in range(4):
                    slot = g * 512 + 128 * i + 32 * j
                    qx = rows_v[pl.ds(slot, 16)]
                    qy = rows_v[pl.ds(slot + 16, 16)]
                    rx = rx + wy[j] * qx
                    ry = ry + wy[j] * qy
                mx = mx + wx[i] * rx
                my = my + wx[i] * ry
            ex = c1x_v[pl.ds(g * 16, 16)] - mx
            ey = c1y_v[pl.ds(g * 16, 16)] - my
            return a + ex * ex + ey * ey

        return lax.fori_loop(0, ngroups, comp_one, acc, unroll=False)

    def chunk_body(k, acc):
        return process_chunk(wid * PER_W + k * CHUNK, GROUPS, acc)

    acc = lax.fori_loop(0, NCHUNKS, chunk_body,
                        jnp.zeros((16,), jnp.float32), unroll=False)
    ntail = jnp.where(wid == NW - 1, 1, 0)
    acc = lax.fori_loop(
        0, ntail,
        lambda k, a: process_chunk(NW * PER_W, TAIL_GROUPS, a),
        acc, unroll=False)
    out_v[...] = acc
    pltpu.sync_copy(out_v, out_hbm.at[wid])


@jax.jit
def _run(c1x, c1y, c2x, c2y, rr, cc, tab):
    mesh = plsc.VectorSubcoreMesh(core_axis_name="c", subcore_axis_name="s")
    f = pl.kernel(
        _body,
        out_type=jax.ShapeDtypeStruct((NW, 16), jnp.float32),
        mesh=mesh,
        scratch_types=[
            pltpu.VMEM((CHUNK,), jnp.float32),
            pltpu.VMEM((CHUNK,), jnp.float32),
            pltpu.VMEM((CHUNK,), jnp.float32),
            pltpu.VMEM((CHUNK,), jnp.float32),
            pltpu.VMEM((CHUNK,), jnp.int32),
            pltpu.VMEM((CHUNK,), jnp.int32),
            pltpu.VMEM((GROUPS * 512,), jnp.int32),
            pltpu.VMEM((GROUPS * 512,), jnp.float32),
            pltpu.VMEM((16,), jnp.float32),
            pltpu.SemaphoreType.DMA,
        ],
    )
    partials = f(c1x, c1y, c2x, c2y, rr, cc, tab)
    return jnp.sum(partials)


def kernel(ch1, ch2, CP_locs, CP_idx):
    return _run(ch1[:, 0], ch1[:, 1], ch2[:, 0], ch2[:, 1],
                CP_idx[:, 0], CP_idx[:, 1], CP_locs.reshape(-1))
